# baseline (speedup 1.0000x reference)
import functools

import jax
import jax.numpy as jnp
from jax import lax
from jax.experimental import pallas as pl
from jax.experimental.pallas import tpu as pltpu
from jax.experimental.pallas import tpu_sc as plsc

_LANES = 16
_NUM_SUBCORES = 16
_NUM_CORES = 2
_NUM_WORKERS = _NUM_SUBCORES * _NUM_CORES
_TILE_R = 8
_TILE_C = 128


def _gather_body(n_rows, n_cols, per_tile, out_hbm, y_hbm, part_hbm,
                 y_v, cache_v, acc_v, sem):
  wid = lax.axis_index("s") * _NUM_CORES + lax.axis_index("c")
  base = pl.multiple_of(wid * per_tile, per_tile)

  pltpu.sync_copy(y_hbm.at[pl.ds(base, per_tile)], y_v)

  copies = []
  for j in range(per_tile // _LANES):
    yv = y_v[pl.ds(j * _LANES, _LANES)]
    for k in range(_LANES):
      s = j * _LANES + k
      rb = pl.multiple_of(base + (s // 8) * 8, 8)
      cb = pl.multiple_of(yv[k] & ~(_TILE_C - 1), _TILE_C)
      copies.append(pltpu.async_copy(
          out_hbm.at[pl.ds(rb, _TILE_R), pl.ds(cb, _TILE_C)],
          cache_v.at[s], sem))
  for cp in copies:
    cp.wait()

  lane = lax.iota(jnp.int32, _LANES)
  acc = jnp.zeros((_LANES,), jnp.float32)
  for j in range(per_tile // _LANES):
    yv = y_v[pl.ds(j * _LANES, _LANES)]
    acc = acc + plsc.load_gather(
        cache_v,
        [j * _LANES + lane, lane & (_TILE_R - 1), yv & (_TILE_C - 1)])
  acc_v[...] = acc
  pltpu.sync_copy(acc_v, part_hbm.at[wid])


def _reduce_body(part_hbm, out_hbm, all_v, res_v):
  wid = lax.axis_index("s") * _NUM_CORES + lax.axis_index("c")

  @pl.when(wid == 0)
  def _():
    pltpu.sync_copy(part_hbm, all_v)
    lane = lax.iota(jnp.int32, _LANES)
    acc = jnp.zeros((_LANES,), jnp.float32)
    for t in range(_NUM_WORKERS):
      acc = acc + all_v[t]
    for k in (8, 4, 2, 1):
      res_v[...] = acc
      acc = acc + plsc.load_gather(res_v, [(lane + k) & (_LANES - 1)])
    res_v[...] = acc
    pltpu.sync_copy(res_v, out_hbm)


@functools.partial(jax.jit, static_argnums=(2, 3))
def _relevant_sum(out2d, y, n_rows, n_cols):
  per_tile = n_rows // _NUM_WORKERS
  mesh = plsc.VectorSubcoreMesh(
      core_axis_name="c", subcore_axis_name="s", num_cores=_NUM_CORES)

  partials = pl.kernel(
      functools.partial(_gather_body, n_rows, n_cols, per_tile),
      out_type=jax.ShapeDtypeStruct((_NUM_WORKERS, _LANES), jnp.float32),
      mesh=mesh,
      compiler_params=pltpu.CompilerParams(needs_layout_passes=False),
      scratch_types=[
          pltpu.VMEM((per_tile,), jnp.int32),
          pltpu.VMEM((per_tile, _TILE_R, _TILE_C), jnp.float32),
          pltpu.VMEM((_LANES,), jnp.float32),
          pltpu.SemaphoreType.DMA,
      ],
  )(out2d, y)

  res = pl.kernel(
      _reduce_body,
      out_type=jax.ShapeDtypeStruct((_LANES,), jnp.float32),
      mesh=mesh,
      compiler_params=pltpu.CompilerParams(needs_layout_passes=False),
      scratch_types=[
          pltpu.VMEM((_NUM_WORKERS, _LANES), jnp.float32),
          pltpu.VMEM((_LANES,), jnp.float32),
      ],
  )(partials)
  return res[0]


def kernel(out, y):
  n_rows, n_cols = out.shape
  return _relevant_sum(out, y.astype(jnp.int32), n_rows, n_cols)


# speedup vs baseline: 1.2036x; 1.2036x over previous
import functools

import jax
import jax.numpy as jnp
from jax import lax
from jax.experimental import pallas as pl
from jax.experimental.pallas import tpu as pltpu
from jax.experimental.pallas import tpu_sc as plsc

_LANES = 16
_NUM_SUBCORES = 16
_NUM_CORES = 2
_NUM_WORKERS = _NUM_SUBCORES * _NUM_CORES
_TILE_R = 8
_TILE_C = 128


def _gather_body(n_rows, n_cols, per_tile, out_hbm, y_hbm, part_hbm,
                 y_v, cache_v, acc_v, sem):
  wid = lax.axis_index("s") * _NUM_CORES + lax.axis_index("c")
  base = pl.multiple_of(wid * per_tile, per_tile)

  pltpu.sync_copy(y_hbm.at[pl.ds(base, per_tile)], y_v)

  copies = []
  for j in range(per_tile // _LANES):
    yv = y_v[pl.ds(j * _LANES, _LANES)]
    for k in range(_LANES):
      s = j * _LANES + k
      blk = base // _TILE_R + s // _TILE_R
      cb = pl.multiple_of(yv[k] & ~(_TILE_C - 1), _TILE_C)
      copies.append(pltpu.async_copy(
          out_hbm.at[blk, :, pl.ds(cb, _TILE_C)],
          cache_v.at[s], sem))
  for cp in copies:
    cp.wait()

  lane = lax.iota(jnp.int32, _LANES)
  acc = jnp.zeros((_LANES,), jnp.float32)
  for j in range(per_tile // _LANES):
    yv = y_v[pl.ds(j * _LANES, _LANES)]
    acc = acc + plsc.load_gather(
        cache_v,
        [j * _LANES + lane, lane & (_TILE_R - 1), yv & (_TILE_C - 1)])
  acc_v[...] = acc
  pltpu.sync_copy(acc_v, part_hbm.at[wid])


def _reduce_body(part_hbm, out_hbm, all_v, res_v):
  wid = lax.axis_index("s") * _NUM_CORES + lax.axis_index("c")

  @pl.when(wid == 0)
  def _():
    pltpu.sync_copy(part_hbm, all_v)
    lane = lax.iota(jnp.int32, _LANES)
    acc = jnp.zeros((_LANES,), jnp.float32)
    for t in range(_NUM_WORKERS):
      acc = acc + all_v[t]
    for k in (8, 4, 2, 1):
      res_v[...] = acc
      acc = acc + plsc.load_gather(res_v, [(lane + k) & (_LANES - 1)])
    res_v[...] = acc
    pltpu.sync_copy(res_v, out_hbm)


@functools.partial(jax.jit, static_argnums=(2, 3))
def _relevant_sum(out2d, y, n_rows, n_cols):
  per_tile = n_rows // _NUM_WORKERS
  mesh = plsc.VectorSubcoreMesh(
      core_axis_name="c", subcore_axis_name="s", num_cores=_NUM_CORES)

  partials = pl.kernel(
      functools.partial(_gather_body, n_rows, n_cols, per_tile),
      out_type=jax.ShapeDtypeStruct((_NUM_WORKERS, _LANES), jnp.float32),
      mesh=mesh,
      compiler_params=pltpu.CompilerParams(needs_layout_passes=False),
      scratch_types=[
          pltpu.VMEM((per_tile,), jnp.int32),
          pltpu.VMEM((per_tile, _TILE_R, _TILE_C), jnp.float32),
          pltpu.VMEM((_LANES,), jnp.float32),
          pltpu.SemaphoreType.DMA,
      ],
  )(out2d.reshape(n_rows // _TILE_R, _TILE_R, n_cols), y)

  res = pl.kernel(
      _reduce_body,
      out_type=jax.ShapeDtypeStruct((_LANES,), jnp.float32),
      mesh=mesh,
      compiler_params=pltpu.CompilerParams(needs_layout_passes=False),
      scratch_types=[
          pltpu.VMEM((_NUM_WORKERS, _LANES), jnp.float32),
          pltpu.VMEM((_LANES,), jnp.float32),
      ],
  )(partials)
  return res[0]


def kernel(out, y):
  n_rows, n_cols = out.shape
  return _relevant_sum(out, y.astype(jnp.int32), n_rows, n_cols)


# speedup vs baseline: 1.2045x; 1.0007x over previous
import functools

import jax
import jax.numpy as jnp
from jax import lax
from jax.experimental import pallas as pl
from jax.experimental.pallas import tpu as pltpu
from jax.experimental.pallas import tpu_sc as plsc

_LANES = 16
_NUM_SUBCORES = 16
_NUM_CORES = 1
_NUM_WORKERS = _NUM_SUBCORES * _NUM_CORES
_TILE_R = 8
_TILE_C = 128


def _gather_body(n_rows, n_cols, per_tile, out_hbm, y_hbm, part_hbm,
                 y_v, cache_v, acc_v, sem):
  wid = lax.axis_index("s") * _NUM_CORES + lax.axis_index("c")
  base = pl.multiple_of(wid * per_tile, per_tile)

  pltpu.sync_copy(y_hbm.at[pl.ds(base, per_tile)], y_v)

  copies = []
  for j in range(per_tile // _LANES):
    yv = y_v[pl.ds(j * _LANES, _LANES)]
    for k in range(_LANES):
      s = j * _LANES + k
      blk = base // _TILE_R + s // _TILE_R
      cb = pl.multiple_of(yv[k] & ~(_TILE_C - 1), _TILE_C)
      copies.append(pltpu.async_copy(
          out_hbm.at[blk, :, pl.ds(cb, _TILE_C)],
          cache_v.at[s], sem))
  for cp in copies:
    cp.wait()

  lane = lax.iota(jnp.int32, _LANES)
  acc = jnp.zeros((_LANES,), jnp.float32)
  for j in range(per_tile // _LANES):
    yv = y_v[pl.ds(j * _LANES, _LANES)]
    acc = acc + plsc.load_gather(
        cache_v,
        [j * _LANES + lane, lane & (_TILE_R - 1), yv & (_TILE_C - 1)])
  acc_v[...] = acc
  pltpu.sync_copy(acc_v, part_hbm.at[wid])


def _reduce_body(part_hbm, out_hbm, all_v, res_v):
  wid = lax.axis_index("s") * _NUM_CORES + lax.axis_index("c")

  @pl.when(wid == 0)
  def _():
    pltpu.sync_copy(part_hbm, all_v)
    lane = lax.iota(jnp.int32, _LANES)
    acc = jnp.zeros((_LANES,), jnp.float32)
    for t in range(_NUM_WORKERS):
      acc = acc + all_v[t]
    for k in (8, 4, 2, 1):
      res_v[...] = acc
      acc = acc + plsc.load_gather(res_v, [(lane + k) & (_LANES - 1)])
    res_v[...] = acc
    pltpu.sync_copy(res_v, out_hbm)


@functools.partial(jax.jit, static_argnums=(2, 3))
def _relevant_sum(out2d, y, n_rows, n_cols):
  per_tile = n_rows // _NUM_WORKERS
  mesh = plsc.VectorSubcoreMesh(
      core_axis_name="c", subcore_axis_name="s", num_cores=_NUM_CORES)

  partials = pl.kernel(
      functools.partial(_gather_body, n_rows, n_cols, per_tile),
      out_type=jax.ShapeDtypeStruct((_NUM_WORKERS, _LANES), jnp.float32),
      mesh=mesh,
      compiler_params=pltpu.CompilerParams(needs_layout_passes=False),
      scratch_types=[
          pltpu.VMEM((per_tile,), jnp.int32),
          pltpu.VMEM((per_tile, _TILE_R, _TILE_C), jnp.float32),
          pltpu.VMEM((_LANES,), jnp.float32),
          pltpu.SemaphoreType.DMA,
      ],
  )(out2d.reshape(n_rows // _TILE_R, _TILE_R, n_cols), y)

  res = pl.kernel(
      _reduce_body,
      out_type=jax.ShapeDtypeStruct((_LANES,), jnp.float32),
      mesh=mesh,
      compiler_params=pltpu.CompilerParams(needs_layout_passes=False),
      scratch_types=[
          pltpu.VMEM((_NUM_WORKERS, _LANES), jnp.float32),
          pltpu.VMEM((_LANES,), jnp.float32),
      ],
  )(partials)
  return res[0]


def kernel(out, y):
  n_rows, n_cols = out.shape
  return _relevant_sum(out, y.astype(jnp.int32), n_rows, n_cols)
